# Initial kernel scaffold; baseline (speedup 1.0000x reference)
import jax
import jax.numpy as jnp
from jax.experimental import pallas as pl


def kernel(env_idx, W_adj, env_deltas):
    raise NotImplementedError("write your pallas kernel here")


# speedup vs baseline: 1.3957x; 1.3957x over previous
import jax
import jax.numpy as jnp
from jax.experimental import pallas as pl
from jax.experimental.pallas import tpu as pltpu

_D = 128
_N = 100
_B = 1024
_BB = 8
_TEMP = 1.0


def _body(env_idx_ref, w_ref, deltas_ref, a_ref, wb_ref):
    i = pl.program_id(0)
    w = w_ref[...]
    row = jax.lax.broadcasted_iota(jnp.int32, (_D, _D), 0)
    col = jax.lax.broadcasted_iota(jnp.int32, (_D, _D), 1)
    mask = jnp.where(row == col, 0.0, 1.0)
    base = i * _BB
    for j in range(_BB):
        e = env_idx_ref[base + j]
        valid = e < _N
        idx = jnp.clip(e, 0, _N - 1)
        delta = deltas_ref[idx]
        wb = jnp.where(valid, 1.0, 0.0) * delta + w
        wb_ref[j] = wb
        a_ref[j] = jax.nn.sigmoid(wb * (1.0 / _TEMP)) * mask


@jax.jit
def _run(env_idx, W_adj, env_deltas):
    grid = (_B // _BB,)
    out_shape = (
        jax.ShapeDtypeStruct((_B, _D, _D), jnp.float32),
        jax.ShapeDtypeStruct((_B, _D, _D), jnp.float32),
    )
    return pl.pallas_call(
        _body,
        grid=grid,
        in_specs=[
            pl.BlockSpec(memory_space=pltpu.SMEM),
            pl.BlockSpec((_D, _D), lambda i: (0, 0)),
            pl.BlockSpec((_N, _D, _D), lambda i: (0, 0, 0)),
        ],
        out_specs=[
            pl.BlockSpec((_BB, _D, _D), lambda i: (i, 0, 0)),
            pl.BlockSpec((_BB, _D, _D), lambda i: (i, 0, 0)),
        ],
        out_shape=out_shape,
    )(env_idx, W_adj, env_deltas)


def kernel(env_idx, W_adj, env_deltas):
    return _run(env_idx, W_adj, env_deltas)
